# baseline (device time: 103998 ns/iter reference)
import jax
import jax.numpy as jnp
from jax import lax
from jax.experimental import pallas as pl
from jax.experimental.pallas import tpu as pltpu

N_DEV = 4
SQ = 1024
SQ2 = 512
D = 1024
HG = 8
HH = 4
DH = 128
DHALF = HH * DH
BLK = 64
SCALE = 0.08838834764831843
F32 = jnp.float32
BF16 = jnp.bfloat16

_CompilerParams = getattr(pltpu, "CompilerParams", None) or getattr(
    pltpu, "TPUCompilerParams"
)


def kernel(x, Wq, K_ext, V_ext, Wo):
    def body(x_ref, wq_ref, k_hbm, v_hbm, wo_ref, out_ref,
             qA, oA, qB, oB, xbf_ref, ctx_ref, kv_ref,
             kv_sems, send_sems, recv_sems):
        my = lax.axis_index("i")
        left = lax.rem(my + N_DEV - 1, N_DEV)
        right = lax.rem(my + 1, N_DEV)
        opp = lax.rem(my + 2, N_DEV)

        barrier = pltpu.get_barrier_semaphore()
        for nbr in (left, right):
            pl.semaphore_signal(barrier, inc=1, device_id=(nbr,),
                                device_id_type=pl.DeviceIdType.MESH)
        pl.semaphore_wait(barrier, 2)

        xbf_ref[...] = (x_ref[0] * SCALE).astype(BF16)
        qA[my] = wq_ref[:, :DHALF].astype(BF16)
        qB[my] = wq_ref[:, DHALF:].astype(BF16)
        oA[my] = wo_ref[:DHALF, :].astype(BF16)
        oB[my] = wo_ref[DHALF:, :].astype(BF16)

        qb = lax.broadcasted_iota(jnp.int32, (SQ2, SQ2), 0) // BLK
        kb = lax.broadcasted_iota(jnp.int32, (SQ2, SQ2), 1) // BLK
        mask = kb <= qb

        def send_pair(dst, link, j, bufq, bufo, slot):
            out = []
            for t, buf in ((0, bufq), (1, bufo)):
                r = pltpu.make_async_remote_copy(
                    src_ref=buf.at[slot],
                    dst_ref=buf.at[slot],
                    send_sem=send_sems.at[link, j, t],
                    recv_sem=recv_sems.at[link, j, t],
                    device_id=(dst,),
                    device_id_type=pl.DeviceIdType.MESH,
                )
                r.start()
                out.append(r)
            return out

        def wait_pair(link, j, bufq, bufo, slot):
            for t, buf in ((0, bufq), (1, bufo)):
                pltpu.make_async_remote_copy(
                    src_ref=buf.at[slot],
                    dst_ref=buf.at[slot],
                    send_sem=send_sems.at[link, j, t],
                    recv_sem=recv_sems.at[link, j, t],
                    device_id=(left,),
                    device_id_type=pl.DeviceIdType.MESH,
                ).wait_recv()

        kv_plan = [
            (my, 0, my, HH),
            (left, 0, right, HH),
            (left, HH, right, 0),
            (opp, 0, opp, HH),
        ]

        def start_kv(step):
            p = step % 2
            ga, offa, gb, offb = kv_plan[step]
            copies = []
            for i, (g, off) in enumerate(
                    ((ga, offa), (ga, offa), (gb, offb), (gb, offb))):
                src = (k_hbm if i % 2 == 0 else v_hbm)
                copies.append(pltpu.make_async_copy(
                    src.at[my, :, pl.ds(g * HG + off, HH), :],
                    kv_ref.at[p, i], kv_sems.at[p, i]))
            for c in copies:
                c.start()
            return copies

        ones_col = jnp.ones((SQ2, 1), BF16)

        def compute_half(wq_h, wo_h, p, kv_k, kv_v):
            q = lax.dot(xbf_ref[...], wq_h,
                        preferred_element_type=F32).astype(BF16)
            dot_t = lambda a, b: lax.dot_general(
                a, b, (((1,), (1,)), ((), ())),
                preferred_element_type=F32).astype(BF16)
            rowsum = lambda w: lax.dot(w, ones_col,
                                       preferred_element_type=F32)
            zero = jnp.zeros((), BF16)
            for h in range(HH):
                qh_u = q[:SQ2, h * DH:(h + 1) * DH]
                qh_l = q[SQ2:, h * DH:(h + 1) * DH]
                kh = kv_ref[p, kv_k, :, h, :].astype(BF16)
                vh = kv_ref[p, kv_v, :, h, :].astype(BF16)
                w_u = jnp.where(mask, jnp.exp(dot_t(qh_u, kh[:SQ2])), zero)
                ctx_u = lax.dot(w_u, vh[:SQ2],
                                preferred_element_type=F32) / rowsum(w_u)
                s_l = dot_t(qh_l, kh)
                w_ll = jnp.exp(s_l[:, :SQ2])
                w_lr = jnp.where(mask, jnp.exp(s_l[:, SQ2:]), zero)
                den_l = rowsum(w_ll) + rowsum(w_lr)
                ctx_l = (lax.dot(w_ll, vh[:SQ2],
                                 preferred_element_type=F32)
                         + lax.dot(w_lr, vh[SQ2:],
                                   preferred_element_type=F32)) / den_l
                ctx_ref[:SQ2, h * DH:(h + 1) * DH] = ctx_u.astype(BF16)
                ctx_ref[SQ2:, h * DH:(h + 1) * DH] = ctx_l.astype(BF16)
            return lax.dot(ctx_ref[...], wo_h, preferred_element_type=F32)

        sends = []
        sends += send_pair(right, 0, 0, qA, oA, my)
        sends += send_pair(left, 1, 0, qB, oB, my)
        sends += send_pair(right, 0, 1, qB, oB, my)
        sends += send_pair(left, 1, 1, qA, oA, my)
        kv_copies = start_kv(0)
        for c in kv_copies:
            c.wait()
        kv_copies = start_kv(1)
        out_ref[0] = compute_half(qA[my], oA[my], 0, 0, 1)
        out_ref[0] += compute_half(qB[my], oB[my], 0, 2, 3)

        wait_pair(0, 0, qA, oA, left)
        wait_pair(1, 0, qB, oB, right)
        sends += send_pair(right, 0, 2, qA, oA, left)
        sends += send_pair(left, 1, 2, qB, oB, right)
        for c in kv_copies:
            c.wait()
        kv_copies = start_kv(2)
        out_ref[0] += compute_half(qA[left], oA[left], 1, 0, 1)
        out_ref[0] += compute_half(qB[right], oB[right], 1, 2, 3)

        wait_pair(0, 1, qB, oB, left)
        wait_pair(1, 1, qA, oA, right)
        for c in kv_copies:
            c.wait()
        kv_copies = start_kv(3)
        out_ref[0] += compute_half(qB[left], oB[left], 0, 0, 1)
        out_ref[0] += compute_half(qA[right], oA[right], 0, 2, 3)

        wait_pair(0, 2, qA, oA, opp)
        wait_pair(1, 2, qB, oB, opp)
        for c in kv_copies:
            c.wait()
        out_ref[0] += compute_half(qA[opp], oA[opp], 1, 0, 1)
        out_ref[0] += compute_half(qB[opp], oB[opp], 1, 2, 3)

        for r in sends:
            r.wait_send()

    return pl.pallas_call(
        body,
        out_shape=jax.ShapeDtypeStruct((1, SQ, D), jnp.float32),
        in_specs=[
            pl.BlockSpec(memory_space=pltpu.MemorySpace.VMEM),
            pl.BlockSpec(memory_space=pltpu.MemorySpace.VMEM),
            pl.BlockSpec(memory_space=pl.ANY),
            pl.BlockSpec(memory_space=pl.ANY),
            pl.BlockSpec(memory_space=pltpu.MemorySpace.VMEM),
        ],
        out_specs=pl.BlockSpec(memory_space=pltpu.MemorySpace.VMEM),
        scratch_shapes=[
            pltpu.VMEM((N_DEV, D, DHALF), BF16),
            pltpu.VMEM((N_DEV, DHALF, D), BF16),
            pltpu.VMEM((N_DEV, D, DHALF), BF16),
            pltpu.VMEM((N_DEV, DHALF, D), BF16),
            pltpu.VMEM((SQ, D), BF16),
            pltpu.VMEM((SQ, DHALF), BF16),
            pltpu.VMEM((2, 4, SQ, HH, DH), F32),
            pltpu.SemaphoreType.DMA((2, 4)),
            pltpu.SemaphoreType.DMA((2, 3, 2)),
            pltpu.SemaphoreType.DMA((2, 3, 2)),
        ],
        compiler_params=_CompilerParams(
            collective_id=0, vmem_limit_bytes=100 * 1024 * 1024),
    )(x, Wq, K_ext, V_ext, Wo)
